# baseline (speedup 1.0000x reference)
import functools

import numpy as np

import jax
import jax.numpy as jnp
from jax import lax
from jax.experimental import pallas as pl
from jax.experimental.pallas import tpu as pltpu
from jax.experimental.pallas import tpu_sc as plsc

_N_TOKENS = 16384
_D_MODEL = 4096
_N_EXPERTS = 64
_TOP_K = 8
_SCALE = 2.5
_MM_BLOCK = 1024

_NW = 32
_N_CHUNKS = 1
_CHUNK = _N_TOKENS // _N_CHUNKS
_TPW = _CHUNK // _NW

_INT_MIN = np.int32(-2147483648)


def _gates_body(x_ref, w_ref, o_ref):
    gates = jax.lax.dot_general(
        x_ref[...], w_ref[...], (((1,), (1,)), ((), ())),
        preferred_element_type=jnp.float32)
    o_ref[...] = jax.nn.sigmoid(gates)


def _tc_orig(x, weight):
    n_tokens, d_model = x.shape
    n_experts = weight.shape[0]
    return pl.pallas_call(
        _gates_body,
        grid=(n_tokens // _MM_BLOCK,),
        in_specs=[
            pl.BlockSpec((_MM_BLOCK, d_model), lambda i: (i, 0)),
            pl.BlockSpec((n_experts, d_model), lambda i: (0, 0)),
        ],
        out_specs=pl.BlockSpec((_MM_BLOCK, n_experts), lambda i: (i, 0)),
        out_shape=jax.ShapeDtypeStruct((n_tokens, n_experts), jnp.float32),
        compiler_params=pltpu.CompilerParams(
            dimension_semantics=("parallel",)),
    )(x, weight)


def _mono(v):
    b = plsc.bitcast(v, jnp.int32)
    return jnp.where(b < 0, _INT_MIN - b, b)


def _sorted_keys(k):
    res = plsc.sort_key_val(k, k)
    if isinstance(res, (tuple, list)):
        return res[0]
    return res


def _sc_router_body(orig_hbm, bias_hbm, inds_hbm, sel_hbm, slab, biasv,
                    sscr, t2scr, dscr, csscr, oinds, osel):
    wid = lax.axis_index("s") * 2 + lax.axis_index("c")
    base = wid * _TPW
    pltpu.sync_copy(orig_hbm.at[pl.ds(base * _N_EXPERTS,
                                      _TPW * _N_EXPERTS)], slab)
    pltpu.sync_copy(bias_hbm, biasv)

    lane = lax.iota(jnp.int32, 16)
    lane8 = lane & 7
    hidx = lane8 * 8
    zids = jnp.where(lane < 15, lane - 7, 9)
    cids = jnp.where(lane < 8, hidx, zids)
    low = lane < 8
    biases = [biasv[pl.ds(16 * j, 16)] for j in range(4)]
    seg_hi = jnp.where(low, 0, jnp.int32(1) << 29)
    t2idx = 16 * (lane8 // 2) + 8 * (lane8 % 2) + 6

    def token_body(t, carry):
        slot = (t & 1) * _N_EXPERTS
        trow = lane * 0 + t * _N_EXPERTS
        for j in range(4):
            v = plsc.load_gather(slab, [trow + (lane + 16 * j)]) + biases[j]
            sscr[pl.ds(slot + 16 * j, 16)] = v
            pkey = (_mono(v) >> 3) + seg_hi
            t2scr[pl.ds(slot + 16 * j, 16)] = plsc.sort_key_val(pkey, v)[1]

        gs = (plsc.load_gather(t2scr, [slot + t2idx])
              + plsc.load_gather(t2scr, [slot + t2idx + 1]))
        gkey = (_mono(gs) & jnp.int32(-8)) | lane8
        gkey = jnp.where(low, gkey, jnp.int32(2147483647))
        sgk = _sorted_keys(gkey)
        plsc.store_scatter(dscr, [(t & 1) * 8 + (sgk & 7)],
                           jnp.where(lane < 4, 1.0, 0.0), mask=low)

        hvals = plsc.load_gather(sscr, [slot + hidx])
        dflag = plsc.load_gather(dscr, [(t & 1) * 8 + lane8])
        cvals = jnp.where(low & (dflag > 0.5), hvals, 0.0)
        ckey = (_mono(cvals) & jnp.int32(-64)) | cids
        sck = _sorted_keys(ckey)
        ids = sck & 63

        ovals = plsc.load_gather(slab, [trow + ids])
        svals = jnp.where(low, ovals, 0.0)
        csscr[pl.ds((t & 1) * 16, 16)] = plsc.cumsum(svals)
        denom = plsc.load_gather(csscr, [lane * 0 + (t & 1) * 16 + 15])
        selv = svals / (denom + 1e-20) * _SCALE

        orow = lane * 0 + t * _TOP_K
        plsc.store_scatter(oinds, [orow + lane], ids, mask=low)
        plsc.store_scatter(osel, [orow + lane], selv, mask=low)
        return carry

    lax.fori_loop(0, _TPW, token_body, 0)
    pltpu.sync_copy(oinds, inds_hbm.at[pl.ds(base * _TOP_K,
                                             _TPW * _TOP_K)])
    pltpu.sync_copy(osel, sel_hbm.at[pl.ds(base * _TOP_K,
                                           _TPW * _TOP_K)])


@functools.cache
def _sc_router():
    mesh = plsc.VectorSubcoreMesh(core_axis_name="c", subcore_axis_name="s")
    return pl.kernel(
        _sc_router_body,
        mesh=mesh,
        compiler_params=pltpu.CompilerParams(needs_layout_passes=False),
        out_type=[
            jax.ShapeDtypeStruct((_CHUNK * _TOP_K,), jnp.int32),
            jax.ShapeDtypeStruct((_CHUNK * _TOP_K,), jnp.float32),
        ],
        scratch_types=[
            pltpu.VMEM((_TPW * _N_EXPERTS,), jnp.float32),
            pltpu.VMEM((_N_EXPERTS,), jnp.float32),
            pltpu.VMEM((2 * _N_EXPERTS,), jnp.float32),
            pltpu.VMEM((2 * _N_EXPERTS,), jnp.float32),
            pltpu.VMEM((16,), jnp.float32),
            pltpu.VMEM((32,), jnp.float32),
            pltpu.VMEM((_TPW * _TOP_K,), jnp.int32),
            pltpu.VMEM((_TPW * _TOP_K,), jnp.float32),
        ],
    )


def kernel(x, weight, e_score_correction_bias):
    router = _sc_router()
    inds_parts, sel_parts = [], []
    for c in range(_N_CHUNKS):
        xc = x[c * _CHUNK:(c + 1) * _CHUNK]
        orig = _tc_orig(xc, weight)
        inds, sel = router(orig.reshape(-1), e_score_correction_bias)
        inds_parts.append(inds.reshape(_CHUNK, _TOP_K))
        sel_parts.append(sel.reshape(_CHUNK, _TOP_K))
    return (jnp.concatenate(inds_parts, axis=0),
            jnp.concatenate(sel_parts, axis=0))


# speedup vs baseline: 1.0107x; 1.0107x over previous
import functools

import numpy as np

import jax
import jax.numpy as jnp
from jax import lax
from jax.experimental import pallas as pl
from jax.experimental.pallas import tpu as pltpu
from jax.experimental.pallas import tpu_sc as plsc

_N_TOKENS = 16384
_D_MODEL = 4096
_N_EXPERTS = 64
_TOP_K = 8
_SCALE = 2.5
_MM_BLOCK = 1024

_NW = 32
_N_CHUNKS = 1
_CHUNK = _N_TOKENS // _N_CHUNKS
_TPW = _CHUNK // _NW

_INT_MIN = np.int32(-2147483648)


def _gates_body(x_ref, w_ref, o_ref):
    gates = jax.lax.dot_general(
        x_ref[...], w_ref[...], (((1,), (1,)), ((), ())),
        preferred_element_type=jnp.float32)
    o_ref[...] = jax.nn.sigmoid(gates)


def _tc_orig(x, weight):
    n_tokens, d_model = x.shape
    n_experts = weight.shape[0]
    return pl.pallas_call(
        _gates_body,
        grid=(n_tokens // _MM_BLOCK,),
        in_specs=[
            pl.BlockSpec((_MM_BLOCK, d_model), lambda i: (i, 0)),
            pl.BlockSpec((n_experts, d_model), lambda i: (0, 0)),
        ],
        out_specs=pl.BlockSpec((_MM_BLOCK, n_experts), lambda i: (i, 0)),
        out_shape=jax.ShapeDtypeStruct((n_tokens, n_experts), jnp.float32),
        compiler_params=pltpu.CompilerParams(
            dimension_semantics=("parallel",)),
    )(x, weight)


def _mono(v):
    b = plsc.bitcast(v, jnp.int32)
    return jnp.where(b < 0, _INT_MIN - b, b)


def _sorted_keys(k):
    res = plsc.sort_key_val(k, k)
    if isinstance(res, (tuple, list)):
        return res[0]
    return res


def _sc_router_body(orig_hbm, bias_hbm, inds_hbm, sel_hbm, slab, biasv,
                    sscr, t2scr, dscr, csscr, oinds, osel):
    wid = lax.axis_index("s") * 2 + lax.axis_index("c")
    base = wid * _TPW
    pltpu.sync_copy(orig_hbm.at[pl.ds(base * _N_EXPERTS,
                                      _TPW * _N_EXPERTS)], slab)
    pltpu.sync_copy(bias_hbm, biasv)

    lane = lax.iota(jnp.int32, 16)
    lane8 = lane & 7
    hidx = lane8 * 8
    zids = jnp.where(lane < 15, lane - 7, 9)
    cids = jnp.where(lane < 8, hidx, zids)
    low = lane < 8
    biases = [biasv[pl.ds(16 * j, 16)] for j in range(4)]
    seg_hi = jnp.where(low, 0, jnp.int32(1) << 29)
    t2idx = 16 * (lane8 // 2) + 8 * (lane8 % 2) + 6

    def one_token(t, parity):
        slot = parity * _N_EXPERTS
        trow = lane * 0 + t * _N_EXPERTS
        for j in range(4):
            v = plsc.load_gather(slab, [trow + (lane + 16 * j)]) + biases[j]
            sscr[pl.ds(slot + 16 * j, 16)] = v
            pkey = (_mono(v) >> 3) + seg_hi
            t2scr[pl.ds(slot + 16 * j, 16)] = plsc.sort_key_val(pkey, v)[1]

        gs = (plsc.load_gather(t2scr, [slot + t2idx])
              + plsc.load_gather(t2scr, [slot + t2idx + 1]))
        gkey = (_mono(gs) & jnp.int32(-8)) | lane8
        gkey = jnp.where(low, gkey, jnp.int32(2147483647))
        sgk = _sorted_keys(gkey)
        plsc.store_scatter(dscr, [parity * 8 + (sgk & 7)],
                           jnp.where(lane < 4, 1.0, 0.0), mask=low)

        hvals = plsc.load_gather(sscr, [slot + hidx])
        dflag = plsc.load_gather(dscr, [parity * 8 + lane8])
        cvals = jnp.where(low & (dflag > 0.5), hvals, 0.0)
        ckey = (_mono(cvals) & jnp.int32(-64)) | cids
        sck = _sorted_keys(ckey)
        ids = sck & 63

        ovals = plsc.load_gather(slab, [trow + ids])
        svals = jnp.where(low, ovals, 0.0)
        csscr[pl.ds(parity * 16, 16)] = plsc.cumsum(svals)
        denom = plsc.load_gather(csscr, [lane * 0 + parity * 16 + 15])
        selv = svals / (denom + 1e-20) * _SCALE

        orow = lane * 0 + t * _TOP_K
        plsc.store_scatter(oinds, [orow + lane], ids, mask=low)
        plsc.store_scatter(osel, [orow + lane], selv, mask=low)

    def token_pair(i, carry):
        one_token(2 * i, 0)
        one_token(2 * i + 1, 1)
        return carry

    lax.fori_loop(0, _TPW // 2, token_pair, 0)
    pltpu.sync_copy(oinds, inds_hbm.at[pl.ds(base * _TOP_K,
                                             _TPW * _TOP_K)])
    pltpu.sync_copy(osel, sel_hbm.at[pl.ds(base * _TOP_K,
                                           _TPW * _TOP_K)])


@functools.cache
def _sc_router():
    mesh = plsc.VectorSubcoreMesh(core_axis_name="c", subcore_axis_name="s")
    return pl.kernel(
        _sc_router_body,
        mesh=mesh,
        compiler_params=pltpu.CompilerParams(needs_layout_passes=False),
        out_type=[
            jax.ShapeDtypeStruct((_CHUNK * _TOP_K,), jnp.int32),
            jax.ShapeDtypeStruct((_CHUNK * _TOP_K,), jnp.float32),
        ],
        scratch_types=[
            pltpu.VMEM((_TPW * _N_EXPERTS,), jnp.float32),
            pltpu.VMEM((_N_EXPERTS,), jnp.float32),
            pltpu.VMEM((2 * _N_EXPERTS,), jnp.float32),
            pltpu.VMEM((2 * _N_EXPERTS,), jnp.float32),
            pltpu.VMEM((16,), jnp.float32),
            pltpu.VMEM((32,), jnp.float32),
            pltpu.VMEM((_TPW * _TOP_K,), jnp.int32),
            pltpu.VMEM((_TPW * _TOP_K,), jnp.float32),
        ],
    )


def kernel(x, weight, e_score_correction_bias):
    router = _sc_router()
    inds_parts, sel_parts = [], []
    for c in range(_N_CHUNKS):
        xc = x[c * _CHUNK:(c + 1) * _CHUNK]
        orig = _tc_orig(xc, weight)
        inds, sel = router(orig.reshape(-1), e_score_correction_bias)
        inds_parts.append(inds.reshape(_CHUNK, _TOP_K))
        sel_parts.append(sel.reshape(_CHUNK, _TOP_K))
    return (jnp.concatenate(inds_parts, axis=0),
            jnp.concatenate(sel_parts, axis=0))
